# baseline (speedup 1.0000x reference)
import functools

import jax
import jax.numpy as jnp
from jax import lax
from jax.experimental import pallas as pl
from jax.experimental.pallas import tpu as pltpu
from jax.experimental.pallas import tpu_sc as plsc

D_MODEL = 64
CHUNK = 128
LANES = 16


@functools.partial(jax.jit, static_argnums=(2, 3, 4))
def _embed_gather(idx2d, table, n_t, n_b, blocks_per_w):
    mesh = plsc.VectorSubcoreMesh(core_axis_name="c", subcore_axis_name="s")
    num_cores = mesh.num_cores

    @functools.partial(
        pl.kernel,
        out_type=jax.ShapeDtypeStruct((n_t, D_MODEL, n_b), jnp.float32),
        mesh=mesh,
        scratch_types=[
            pltpu.VMEM((blocks_per_w, CHUNK), jnp.int32),
            pltpu.VMEM((CHUNK, D_MODEL), jnp.float32),
            pltpu.VMEM((D_MODEL, CHUNK), jnp.float32),
            pltpu.SemaphoreType.DMA,
        ],
        compiler_params=pltpu.CompilerParams(
            use_tc_tiling_on_sc=False, needs_layout_passes=False
        ),
    )
    def k(idx_hbm, table_hbm, out_hbm, idx_v, rows_v, tout_v, sem):
        wid = lax.axis_index("s") * num_cores + lax.axis_index("c")
        blk0 = wid * blocks_per_w
        pltpu.sync_copy(idx_hbm.at[pl.ds(blk0, blocks_per_w)], idx_v)
        cb_per_t = n_b // CHUNK

        @pl.loop(0, blocks_per_w)
        def _(j):
            beta = blk0 + j
            t = beta // cb_per_t
            cb = beta % cb_per_t
            pltpu.async_copy(table_hbm.at[idx_v.at[j]], rows_v, sem).wait()

            @pl.loop(0, D_MODEL)
            def _(d):
                dvec = jnp.full((LANES,), d, dtype=jnp.int32)
                for g in range(CHUNK // LANES):
                    bidx = lax.iota(jnp.int32, LANES) + g * LANES
                    v = plsc.load_gather(rows_v, [bidx, dvec])
                    tout_v[d, pl.ds(g * LANES, LANES)] = v

            pltpu.sync_copy(tout_v, out_hbm.at[t, :, pl.ds(cb * CHUNK, CHUNK)])

    return k(idx2d, table)


def kernel(tokens, W_E):
    b, t = tokens.shape
    n_rows = b * t
    num_workers = 32
    assert n_rows % (num_workers * CHUNK) == 0
    blocks_per_w = n_rows // (num_workers * CHUNK)
    idx2d = tokens.T.reshape(n_rows // CHUNK, CHUNK).astype(jnp.int32)
    out = _embed_gather(idx2d, W_E, t, b, blocks_per_w)
    return jnp.transpose(out, (2, 0, 1))
